# Initial kernel scaffold; baseline (speedup 1.0000x reference)
import jax
import jax.numpy as jnp
from jax.experimental import pallas as pl


def kernel(graph, edge_index, W1, a_src1, a_tgt1, b1, W2, a_src2, a_tgt2, b2):
    raise NotImplementedError("write your pallas kernel here")


# speedup vs baseline: 7.9082x; 7.9082x over previous
import functools

import jax
import jax.numpy as jnp
from jax import lax
from jax.experimental import pallas as pl
from jax.experimental.pallas import tpu as pltpu
from jax.experimental.pallas import tpu_sc as plsc

N = 10000
NP = 10240
TRASH = N
D = 256
E_IN = 160000
E2 = E_IN + N
CH = 64
EP = 170240
NCHUNK = EP // CH
NC, NS = 2, 16
NW = NC * NS
NEG = 0.2
R = 1024
G = NP // R

_mesh = plsc.VectorSubcoreMesh(core_axis_name="c", subcore_axis_name="s")
_cparams = pltpu.CompilerParams(needs_layout_passes=False)


def _k1_body(H, F, x_ref, w_ref, as_ref, at_ref, proj_ref, s_ref, cm_ref):
    i = pl.program_id(0)
    x = x_ref[...]
    proj = jnp.dot(x, w_ref[...], preferred_element_type=jnp.float32)
    proj_ref[...] = proj
    p3 = proj.reshape(R, H, F)
    ss = jnp.sum(p3 * as_ref[...][None], axis=2)
    st = jnp.sum(p3 * at_ref[...][None], axis=2)
    s_ref[...] = jnp.concatenate([ss, st], axis=1)
    for h in range(H):
        ms = jnp.max(ss[:, h])
        mt = jnp.max(st[:, h])

        @pl.when(i == 0)
        def _():
            cm_ref[0, h] = ms
            cm_ref[0, H + h] = mt

        @pl.when(i > 0)
        def _():
            cm_ref[0, h] = jnp.maximum(cm_ref[0, h], ms)
            cm_ref[0, H + h] = jnp.maximum(cm_ref[0, H + h], mt)


def _proj_scores(x, w, a_s, a_t, H, F):
    Din = x.shape[1]
    Dout = H * F
    return pl.pallas_call(
        functools.partial(_k1_body, H, F),
        grid=(G,),
        in_specs=[
            pl.BlockSpec((R, Din), lambda i: (i, 0)),
            pl.BlockSpec((Din, Dout), lambda i: (0, 0)),
            pl.BlockSpec((H, F), lambda i: (0, 0)),
            pl.BlockSpec((H, F), lambda i: (0, 0)),
        ],
        out_specs=[
            pl.BlockSpec((R, Dout), lambda i: (i, 0)),
            pl.BlockSpec((R, 2 * H), lambda i: (i, 0)),
            pl.BlockSpec(memory_space=pltpu.SMEM),
        ],
        out_shape=[
            jax.ShapeDtypeStruct((NP, Dout), jnp.float32),
            jax.ShapeDtypeStruct((NP, 2 * H), jnp.float32),
            jax.ShapeDtypeStruct((1, 2 * H), jnp.float32),
        ],
    )(x, w, a_s, a_t)


def _k2_body(H, CH2, sA_ref, sB_ref, src_ref, dst_ref, cb_ref,
             pE_ref, part_ref,
             src_v, dst_v, sbuf, tbuf, pbuf, cbuf, accd,
             sem, sem2):
    cid = lax.axis_index("c")
    sid = lax.axis_index("s")
    wid = sid * NC + cid
    iota = lax.iota(jnp.int32, 16)
    HN = H * NP
    NCH2 = EP // CH2
    z16 = jnp.zeros((16,), jnp.float32)

    def zloop(g, _):
        accd[pl.ds(g * 16, 16)] = z16
        return 0
    lax.fori_loop(0, HN // 16, zloop, 0)

    pltpu.sync_copy(cb_ref, cbuf)
    c16 = cbuf[...]
    msk = iota < H
    nper = NCH2 // NW + 1

    def tloop(t, _c):
        ch = wid + t * NW

        @pl.when(ch < NCH2)
        def _():
            base = ch * CH2
            pltpu.sync_copy(src_ref.at[pl.ds(base, CH2)], src_v)
            pltpu.sync_copy(dst_ref.at[pl.ds(base, CH2)], dst_v)
            cpa = pltpu.async_copy(sA_ref.at[src_v], sbuf, sem)
            cpb = pltpu.async_copy(sB_ref.at[dst_v], tbuf, sem2)
            cpa.wait()
            cpb.wait()

            def eloop(g, _):
                d16 = dst_v[pl.ds(g * 16, 16)]
                for j in range(16):
                    e = g * 16 + j
                    x = sbuf[e, pl.ds(0, 16)] + tbuf[e, pl.ds(0, 16)]
                    lr = jnp.where(x > 0, x, NEG * x)
                    p = jnp.exp(lr - c16)
                    pbuf[e, pl.ds(0, 16)] = p
                    idx16 = d16[j] * H + iota
                    plsc.addupdate_scatter(accd, [idx16], p, mask=msk)
                return 0
            lax.fori_loop(0, CH2 // 16, eloop, 0)
            pltpu.sync_copy(pbuf, pE_ref.at[pl.ds(base, CH2)])
        return 0

    lax.fori_loop(0, nper, tloop, 0)
    pltpu.sync_copy(accd, part_ref.at[pl.ds(wid * HN, HN)])


def _edge_p(H, sA, sB, srcp, dstp, cb):
    CH2 = 256 if H == 1 else 64
    body = functools.partial(_k2_body, H, CH2)
    f = pl.kernel(
        body,
        out_type=[
            jax.ShapeDtypeStruct((EP, 16), jnp.float32),
            jax.ShapeDtypeStruct((NW * H * NP,), jnp.float32),
        ],
        mesh=_mesh,
        scratch_types=[
            pltpu.VMEM((CH2,), jnp.int32),
            pltpu.VMEM((CH2,), jnp.int32),
            pltpu.VMEM((CH2, 128), jnp.float32),
            pltpu.VMEM((CH2, 128), jnp.float32),
            pltpu.VMEM((CH2, 16), jnp.float32),
            pltpu.VMEM((16,), jnp.float32),
            pltpu.VMEM((H * NP,), jnp.float32),
            pltpu.SemaphoreType.DMA,
            pltpu.SemaphoreType.DMA,
        ],
        compiler_params=_cparams,
    )
    return f(sA, sB, srcp, dstp, cb)


def _k2c_body(H, part_ref, d_ref, tmp, acc):
    cid = lax.axis_index("c")
    sid = lax.axis_index("s")
    wid = sid * NC + cid
    HN = H * NP
    seg = HN // NW
    off = wid * seg
    z16 = jnp.zeros((16,), jnp.float32)

    def rz(g, _):
        acc[pl.ds(g * 16, 16)] = z16
        return 0
    lax.fori_loop(0, seg // 16, rz, 0)
    for k in range(NW):
        pltpu.sync_copy(part_ref.at[pl.ds(k * HN + off, seg)], tmp)

        def radd(g, _):
            o = pl.ds(g * 16, 16)
            acc[o] = acc[o] + tmp[o]
            return 0
        lax.fori_loop(0, seg // 16, radd, 0)
    pltpu.sync_copy(acc, d_ref.at[pl.ds(off, seg)])


def _denom_reduce(H, part):
    body = functools.partial(_k2c_body, H)
    HN = H * NP
    f = pl.kernel(
        body,
        out_type=[jax.ShapeDtypeStruct((HN,), jnp.float32)],
        mesh=_mesh,
        scratch_types=[
            pltpu.VMEM((HN // NW,), jnp.float32),
            pltpu.VMEM((HN // NW,), jnp.float32),
        ],
        compiler_params=_cparams,
    )
    return f(part)[0]


CHB = 256


def _k2b_body(H, pE_ref, dst_ref, dp_ref, aE_ref,
              dst_v, pbuf, db, sem):
    cid = lax.axis_index("c")
    sid = lax.axis_index("s")
    wid = sid * NC + cid
    nchb = EP // CHB
    nper = nchb // NW + 1
    eps = jnp.full((16,), 1e-16, jnp.float32)

    def tloop(t, _c):
        ch = wid + t * NW

        @pl.when(ch < nchb)
        def _():
            base = ch * CHB
            pltpu.sync_copy(dst_ref.at[pl.ds(base, CHB)], dst_v)
            cp0 = pltpu.async_copy(dp_ref.at[dst_v], db, sem)
            pltpu.sync_copy(pE_ref.at[pl.ds(base, CHB)], pbuf)
            cp0.wait()

            def eloop(g, _):
                for j in range(16):
                    e = g * 16 + j
                    dsum = db[e, pl.ds(0, 16)] + eps
                    pbuf[e, pl.ds(0, 16)] = pbuf[e, pl.ds(0, 16)] / dsum
                return 0
            lax.fori_loop(0, CHB // 16, eloop, 0)
            pltpu.sync_copy(pbuf, aE_ref.at[pl.ds(base, CHB)])
        return 0

    lax.fori_loop(0, nper, tloop, 0)


def _edge_alpha(H, pE, dstp, dp):
    body = functools.partial(_k2b_body, H)
    f = pl.kernel(
        body,
        out_type=[jax.ShapeDtypeStruct((EP, 16), jnp.float32)],
        mesh=_mesh,
        scratch_types=[
            pltpu.VMEM((CHB,), jnp.int32),
            pltpu.VMEM((CHB, 16), jnp.float32),
            pltpu.VMEM((CHB, 128), jnp.float32),
            pltpu.SemaphoreType.DMA,
        ],
        compiler_params=_cparams,
    )
    return f(pE, dstp, dp)[0]


def _k3_body(H, do_elu, scale, pc_ref, src_ref, dst_ref, aE_ref, bb_ref,
             out_ref,
             src_v, dst_v, abuf, prow, prow2, msgbuf, bbuf, acc,
             sem, sem2):
    cid = lax.axis_index("c")
    sid = lax.axis_index("s")
    z16 = jnp.zeros((16,), jnp.float32)

    def zrow(e, _):
        for k in range(8):
            msgbuf[e, pl.ds(k * 16, 16)] = z16
        return 0
    lax.fori_loop(0, CH, zrow, 0)
    nsl = NP // CH

    def zsl(t, _c):
        s = sid + t * NS

        @pl.when(s < nsl)
        def _():
            pltpu.sync_copy(msgbuf, acc.at[pl.ds(s * CH, CH)])
        return 0
    lax.fori_loop(0, nsl // NS + 1, zsl, 0)
    plsc.subcore_barrier()

    nper = NCHUNK // NS + 1

    def tloop(t, _c):
        ch = sid + t * NS

        @pl.when(ch < NCHUNK)
        def _():
            base = ch * CH
            pltpu.sync_copy(src_ref.at[pl.ds(base, CH)], src_v)
            pltpu.sync_copy(dst_ref.at[pl.ds(base, CH)], dst_v)
            pltpu.sync_copy(aE_ref.at[pl.ds(base, CH)], abuf)
            cp0 = pltpu.async_copy(
                pc_ref.at[cid].at[src_v], prow, sem)
            for h in range(H):
                cur, nxt = (prow, prow2) if h % 2 == 0 else (prow2, prow)
                csem = sem if h % 2 == 0 else sem2
                nsem = sem2 if h % 2 == 0 else sem
                if h == 0:
                    cp0.wait()
                else:
                    pltpu.make_async_copy(
                        pc_ref.at[(h * NC) + cid].at[src_v], cur,
                        csem).wait()
                if h + 1 < H:
                    pltpu.async_copy(
                        pc_ref.at[((h + 1) * NC) + cid].at[src_v], nxt,
                        nsem)

                def eloop(g, _):
                    for j in range(16):
                        e = g * 16 + j
                        a = abuf[e, pl.ds(0, 16)][h]
                        for k in range(8):
                            o = pl.ds(k * 16, 16)
                            msgbuf[e, o] = cur[e, o] * a
                    return 0
                lax.fori_loop(0, CH // 16, eloop, 0)
                pltpu.sync_copy(msgbuf, acc.at[dst_v], add=True)
        return 0

    lax.fori_loop(0, nper, tloop, 0)
    plsc.subcore_barrier()
    pltpu.sync_copy(bb_ref.at[pl.ds(cid * 128, 128)], bbuf)

    def wsl(t, _c):
        s = sid + t * NS

        @pl.when(s < nsl)
        def _():
            pltpu.sync_copy(acc.at[pl.ds(s * CH, CH)], msgbuf)

            def trow(r, _):
                grow = s * CH + r
                keep = jnp.broadcast_to(grow < N, (16,))
                for k in range(8):
                    o = pl.ds(k * 16, 16)
                    v = msgbuf[r, o] * scale + bbuf[o]
                    if do_elu:
                        v = jnp.where(v > 0, v, jnp.exp(v) - 1.0)
                    msgbuf[r, o] = jnp.where(keep, v, z16)
                return 0
            lax.fori_loop(0, CH, trow, 0)
            pltpu.sync_copy(msgbuf, out_ref.at[cid, pl.ds(s * CH, CH)])
        return 0

    lax.fori_loop(0, nsl // NS + 1, wsl, 0)


def _aggregate(H, do_elu, scale, pc3, srcp, dstp, aE, bb):
    body = functools.partial(_k3_body, H, do_elu, scale)
    f = pl.kernel(
        body,
        out_type=[jax.ShapeDtypeStruct((NC, NP, 128), jnp.float32)],
        mesh=_mesh,
        scratch_types=[
            pltpu.VMEM((CH,), jnp.int32),
            pltpu.VMEM((CH,), jnp.int32),
            pltpu.VMEM((CH, 16), jnp.float32),
            pltpu.VMEM((CH, 128), jnp.float32),
            pltpu.VMEM((CH, 128), jnp.float32),
            pltpu.VMEM((CH, 128), jnp.float32),
            pltpu.VMEM((128,), jnp.float32),
            pltpu.VMEM_SHARED((NP, 128), jnp.float32),
            pltpu.SemaphoreType.DMA,
            pltpu.SemaphoreType.DMA,
        ],
        compiler_params=_cparams,
    )
    return f(pc3, srcp, dstp, aE, bb)[0]


def _k4_body(x_ref, b_ref, o_ref):
    x = x_ref[...] + b_ref[...]
    m = jnp.max(x, axis=1, keepdims=True)
    ex = jnp.exp(x - m)
    o_ref[...] = ex / jnp.sum(ex, axis=1, keepdims=True)


def _softmax_out(x, b):
    return pl.pallas_call(
        _k4_body,
        grid=(G,),
        in_specs=[
            pl.BlockSpec((R, D), lambda i: (i, 0)),
            pl.BlockSpec((1, D), lambda i: (0, 0)),
        ],
        out_specs=pl.BlockSpec((R, D), lambda i: (i, 0)),
        out_shape=jax.ShapeDtypeStruct((NP, D), jnp.float32),
    )(x, b)


def _gat_layer(x, srcp, dstp, W, a_s, a_t, bias, H, F, do_elu, scale):
    proj, s2, cm = _proj_scores(x, W, a_s, a_t, H, F)
    c = cm[:, :H] + cm[:, H:]
    c = jnp.where(c > 0, c, NEG * c)
    cb = jnp.zeros((16,), jnp.float32).at[:H].set(c[0])
    sA = jnp.zeros((NP, 128), jnp.float32).at[:, :H].set(s2[:, :H])
    sB = jnp.zeros((NP, 128), jnp.float32).at[:, :H].set(s2[:, H:])
    pE, part = _edge_p(H, sA, sB, srcp, dstp, cb)
    df = _denom_reduce(H, part).reshape(NP, H)
    dp = jnp.zeros((NP, 128), jnp.float32).at[:, :H].set(df)
    aE = _edge_alpha(H, pE, dstp, dp)
    pc3 = proj.reshape(NP, H, NC, 128).transpose(1, 2, 0, 3).reshape(
        H * NC, NP, 128)
    bb = bias if do_elu else jnp.zeros((NC * 128,), jnp.float32)
    outT = _aggregate(H, do_elu, scale, pc3, srcp, dstp, aE, bb)
    return outT.transpose(1, 0, 2).reshape(NP, 256)


def kernel(graph, edge_index, W1, a_src1, a_tgt1, b1, W2, a_src2, a_tgt2, b2):
    xp = jnp.zeros((NP, D), jnp.float32).at[:N].set(graph)
    src0, dst0 = edge_index[0], edge_index[1]
    dstm = jnp.where(src0 != dst0, dst0, TRASH)
    loop = jnp.arange(N, dtype=jnp.int32)
    padi = jnp.zeros((EP - E2,), jnp.int32)
    srcp = jnp.concatenate([src0, loop, padi])
    dstp = jnp.concatenate([dstm, loop, padi + TRASH])

    h1 = _gat_layer(xp, srcp, dstp, W1, a_src1, a_tgt1, b1,
                    1, 256, do_elu=True, scale=1.0)
    agg2 = _gat_layer(h1, srcp, dstp, W2, a_src2, a_tgt2, b2,
                      8, 256, do_elu=False, scale=0.125)
    out = _softmax_out(agg2, b2.reshape(1, D))
    return out[:N]


# speedup vs baseline: 8.1171x; 1.0264x over previous
import functools

import jax
import jax.numpy as jnp
from jax import lax
from jax.experimental import pallas as pl
from jax.experimental.pallas import tpu as pltpu
from jax.experimental.pallas import tpu_sc as plsc

N = 10000
NP = 10240
TRASH = N
D = 256
E_IN = 160000
E2 = E_IN + N
CH = 64
EP = 170240
NCHUNK = EP // CH
NPA = 10048
NC, NS = 2, 16
NW = NC * NS
NEG = 0.2
R = 1024
G = NP // R

_mesh = plsc.VectorSubcoreMesh(core_axis_name="c", subcore_axis_name="s")
_cparams = pltpu.CompilerParams(needs_layout_passes=False)


def _k1_body(H, F, x_ref, w_ref, as_ref, at_ref, proj_ref, s_ref, cm_ref):
    i = pl.program_id(0)
    x = x_ref[...]
    proj = jnp.dot(x, w_ref[...], preferred_element_type=jnp.float32)
    proj_ref[...] = proj
    p3 = proj.reshape(R, H, F)
    ss = jnp.sum(p3 * as_ref[...][None], axis=2)
    st = jnp.sum(p3 * at_ref[...][None], axis=2)
    s_ref[...] = jnp.concatenate([ss, st], axis=1)
    for h in range(H):
        ms = jnp.max(ss[:, h])
        mt = jnp.max(st[:, h])

        @pl.when(i == 0)
        def _():
            cm_ref[0, h] = ms
            cm_ref[0, H + h] = mt

        @pl.when(i > 0)
        def _():
            cm_ref[0, h] = jnp.maximum(cm_ref[0, h], ms)
            cm_ref[0, H + h] = jnp.maximum(cm_ref[0, H + h], mt)


def _proj_scores(x, w, a_s, a_t, H, F):
    Din = x.shape[1]
    Dout = H * F
    return pl.pallas_call(
        functools.partial(_k1_body, H, F),
        grid=(G,),
        in_specs=[
            pl.BlockSpec((R, Din), lambda i: (i, 0)),
            pl.BlockSpec((Din, Dout), lambda i: (0, 0)),
            pl.BlockSpec((H, F), lambda i: (0, 0)),
            pl.BlockSpec((H, F), lambda i: (0, 0)),
        ],
        out_specs=[
            pl.BlockSpec((R, Dout), lambda i: (i, 0)),
            pl.BlockSpec((R, 2 * H), lambda i: (i, 0)),
            pl.BlockSpec(memory_space=pltpu.SMEM),
        ],
        out_shape=[
            jax.ShapeDtypeStruct((NP, Dout), jnp.float32),
            jax.ShapeDtypeStruct((NP, 2 * H), jnp.float32),
            jax.ShapeDtypeStruct((1, 2 * H), jnp.float32),
        ],
    )(x, w, a_s, a_t)


def _k2_body(H, CH2, sA_ref, sB_ref, src_ref, dst_ref, cb_ref,
             pE_ref, part_ref,
             src_v, dst_v, sbuf, tbuf, pbuf, cbuf, accd,
             sem, sem2):
    cid = lax.axis_index("c")
    sid = lax.axis_index("s")
    wid = sid * NC + cid
    iota = lax.iota(jnp.int32, 16)
    HN = H * (NP if H == 1 else NPA)
    NCH2 = EP // CH2
    z16 = jnp.zeros((16,), jnp.float32)

    def zloop(g, _):
        accd[pl.ds(g * 16, 16)] = z16
        return 0
    lax.fori_loop(0, HN // 16, zloop, 0)

    pltpu.sync_copy(cb_ref, cbuf)
    c16 = cbuf[...]
    msk = iota < H
    nper = NCH2 // NW + 1

    def tloop(t, _c):
        ch = wid + t * NW

        @pl.when(ch < NCH2)
        def _():
            base = ch * CH2
            pltpu.sync_copy(src_ref.at[pl.ds(base, CH2)], src_v)
            pltpu.sync_copy(dst_ref.at[pl.ds(base, CH2)], dst_v)
            cpa = pltpu.async_copy(sA_ref.at[src_v], sbuf, sem)
            cpb = pltpu.async_copy(sB_ref.at[dst_v], tbuf, sem2)
            cpa.wait()
            cpb.wait()

            def eloop(g, _):
                d16 = dst_v[pl.ds(g * 16, 16)]
                for j in range(16):
                    e = g * 16 + j
                    x = sbuf[e, pl.ds(0, 16)] + tbuf[e, pl.ds(0, 16)]
                    lr = jnp.where(x > 0, x, NEG * x)
                    p = jnp.exp(lr - c16)
                    pbuf[e, pl.ds(0, 16)] = p
                    idx16 = d16[j] * H + iota
                    plsc.addupdate_scatter(accd, [idx16], p, mask=msk)
                return 0
            lax.fori_loop(0, CH2 // 16, eloop, 0)
            pltpu.sync_copy(pbuf, pE_ref.at[pl.ds(base, CH2)])
        return 0

    lax.fori_loop(0, nper, tloop, 0)
    pltpu.sync_copy(accd, part_ref.at[pl.ds(wid * HN, HN)])


def _edge_p(H, sA, sB, srcp, dstp, cb):
    CH2 = 256 if H == 1 else 128
    body = functools.partial(_k2_body, H, CH2)
    f = pl.kernel(
        body,
        out_type=[
            jax.ShapeDtypeStruct((EP, 16), jnp.float32),
            jax.ShapeDtypeStruct((NW * H * (NP if H == 1 else NPA),),
                                 jnp.float32),
        ],
        mesh=_mesh,
        scratch_types=[
            pltpu.VMEM((CH2,), jnp.int32),
            pltpu.VMEM((CH2,), jnp.int32),
            pltpu.VMEM((CH2, 128), jnp.float32),
            pltpu.VMEM((CH2, 128), jnp.float32),
            pltpu.VMEM((CH2, 16), jnp.float32),
            pltpu.VMEM((16,), jnp.float32),
            pltpu.VMEM((H * (NP if H == 1 else NPA),), jnp.float32),
            pltpu.SemaphoreType.DMA,
            pltpu.SemaphoreType.DMA,
        ],
        compiler_params=_cparams,
    )
    return f(sA, sB, srcp, dstp, cb)


def _k2c_body(H, part_ref, d_ref, tmp, acc):
    cid = lax.axis_index("c")
    sid = lax.axis_index("s")
    wid = sid * NC + cid
    HN = H * (NP if H == 1 else NPA)
    seg = HN // NW
    off = wid * seg
    z16 = jnp.zeros((16,), jnp.float32)

    def rz(g, _):
        acc[pl.ds(g * 16, 16)] = z16
        return 0
    lax.fori_loop(0, seg // 16, rz, 0)
    for k in range(NW):
        pltpu.sync_copy(part_ref.at[pl.ds(k * HN + off, seg)], tmp)

        def radd(g, _):
            o = pl.ds(g * 16, 16)
            acc[o] = acc[o] + tmp[o]
            return 0
        lax.fori_loop(0, seg // 16, radd, 0)
    pltpu.sync_copy(acc, d_ref.at[pl.ds(off, seg)])


def _denom_reduce(H, part):
    body = functools.partial(_k2c_body, H)
    HN = H * (NP if H == 1 else NPA)
    f = pl.kernel(
        body,
        out_type=[jax.ShapeDtypeStruct((HN,), jnp.float32)],
        mesh=_mesh,
        scratch_types=[
            pltpu.VMEM((HN // NW,), jnp.float32),
            pltpu.VMEM((HN // NW,), jnp.float32),
        ],
        compiler_params=_cparams,
    )
    return f(part)[0]


CHB = 256


def _k2b_body(H, pE_ref, dst_ref, dp_ref, aE_ref,
              dst_v, pbuf, db, sem):
    cid = lax.axis_index("c")
    sid = lax.axis_index("s")
    wid = sid * NC + cid
    nchb = EP // CHB
    nper = nchb // NW + 1
    eps = jnp.full((16,), 1e-16, jnp.float32)

    def tloop(t, _c):
        ch = wid + t * NW

        @pl.when(ch < nchb)
        def _():
            base = ch * CHB
            pltpu.sync_copy(dst_ref.at[pl.ds(base, CHB)], dst_v)
            cp0 = pltpu.async_copy(dp_ref.at[dst_v], db, sem)
            pltpu.sync_copy(pE_ref.at[pl.ds(base, CHB)], pbuf)
            cp0.wait()

            def eloop(g, _):
                for j in range(16):
                    e = g * 16 + j
                    dsum = db[e, pl.ds(0, 16)] + eps
                    pbuf[e, pl.ds(0, 16)] = pbuf[e, pl.ds(0, 16)] / dsum
                return 0
            lax.fori_loop(0, CHB // 16, eloop, 0)
            pltpu.sync_copy(pbuf, aE_ref.at[pl.ds(base, CHB)])
        return 0

    lax.fori_loop(0, nper, tloop, 0)


def _edge_alpha(H, pE, dstp, dp):
    body = functools.partial(_k2b_body, H)
    f = pl.kernel(
        body,
        out_type=[jax.ShapeDtypeStruct((EP, 16), jnp.float32)],
        mesh=_mesh,
        scratch_types=[
            pltpu.VMEM((CHB,), jnp.int32),
            pltpu.VMEM((CHB, 16), jnp.float32),
            pltpu.VMEM((CHB, 128), jnp.float32),
            pltpu.SemaphoreType.DMA,
        ],
        compiler_params=_cparams,
    )
    return f(pE, dstp, dp)[0]


def _k3_body(H, do_elu, scale, pc_ref, src_ref, dst_ref, aE_ref, bb_ref,
             out_ref,
             src_v, dst_v, abuf, prow, prow2, msgbuf, msgbuf2, bbuf, acc,
             sem, sem2, semA, semB):
    cid = lax.axis_index("c")
    sid = lax.axis_index("s")
    z16 = jnp.zeros((16,), jnp.float32)

    def zrow(e, _):
        for k in range(8):
            msgbuf[e, pl.ds(k * 16, 16)] = z16
        return 0
    lax.fori_loop(0, CH, zrow, 0)
    nsl = NPA // CH

    def zsl(t, _c):
        s = sid + t * NS

        @pl.when(s < nsl)
        def _():
            pltpu.sync_copy(msgbuf, acc.at[pl.ds(s * CH, CH)])
        return 0
    lax.fori_loop(0, nsl // NS + 1, zsl, 0)
    plsc.subcore_barrier()

    nper = NCHUNK // NS + 1

    def tloop(t, _c):
        ch = sid + t * NS

        @pl.when(ch < NCHUNK)
        def _():
            base = ch * CH
            pltpu.sync_copy(src_ref.at[pl.ds(base, CH)], src_v)
            pltpu.sync_copy(dst_ref.at[pl.ds(base, CH)], dst_v)
            pltpu.sync_copy(aE_ref.at[pl.ds(base, CH)], abuf)
            cp0 = pltpu.async_copy(
                pc_ref.at[cid].at[src_v], prow, sem)
            for h in range(H):
                cur, nxt = (prow, prow2) if h % 2 == 0 else (prow2, prow)
                csem = sem if h % 2 == 0 else sem2
                nsem = sem2 if h % 2 == 0 else sem
                mcur = msgbuf if h % 2 == 0 else msgbuf2
                msem = semA if h % 2 == 0 else semB
                if h == 0:
                    cp0.wait()
                else:
                    pltpu.make_async_copy(
                        pc_ref.at[(h * NC) + cid].at[src_v], cur,
                        csem).wait()
                if h + 1 < H:
                    pltpu.async_copy(
                        pc_ref.at[((h + 1) * NC) + cid].at[src_v], nxt,
                        nsem)
                if h >= 2:
                    pltpu.make_async_copy(
                        mcur, acc.at[dst_v], msem).wait()

                def eloop(g, _):
                    for j in range(16):
                        e = g * 16 + j
                        a = abuf[e, pl.ds(0, 16)][h]
                        for k in range(8):
                            o = pl.ds(k * 16, 16)
                            mcur[e, o] = cur[e, o] * a
                    return 0
                lax.fori_loop(0, CH // 16, eloop, 0)
                pltpu.async_copy(mcur, acc.at[dst_v], msem, add=True)
            if H >= 2:
                h2 = H - 2
                pltpu.make_async_copy(
                    msgbuf if h2 % 2 == 0 else msgbuf2,
                    acc.at[dst_v],
                    semA if h2 % 2 == 0 else semB).wait()
            h1 = H - 1
            pltpu.make_async_copy(
                msgbuf if h1 % 2 == 0 else msgbuf2,
                acc.at[dst_v],
                semA if h1 % 2 == 0 else semB).wait()
        return 0

    lax.fori_loop(0, nper, tloop, 0)
    plsc.subcore_barrier()
    pltpu.sync_copy(bb_ref.at[pl.ds(cid * 128, 128)], bbuf)

    def wsl(t, _c):
        s = sid + t * NS

        @pl.when(s < nsl)
        def _():
            pltpu.sync_copy(acc.at[pl.ds(s * CH, CH)], msgbuf)

            def trow(r, _):
                grow = s * CH + r
                keep = jnp.broadcast_to(grow < N, (16,))
                for k in range(8):
                    o = pl.ds(k * 16, 16)
                    v = msgbuf[r, o] * scale + bbuf[o]
                    if do_elu:
                        v = jnp.where(v > 0, v, jnp.exp(v) - 1.0)
                    msgbuf[r, o] = jnp.where(keep, v, z16)
                return 0
            lax.fori_loop(0, CH, trow, 0)
            pltpu.sync_copy(msgbuf, out_ref.at[cid, pl.ds(s * CH, CH)])
        return 0

    lax.fori_loop(0, nsl // NS + 1, wsl, 0)


def _aggregate(H, do_elu, scale, pc3, srcp, dstp, aE, bb):
    body = functools.partial(_k3_body, H, do_elu, scale)
    f = pl.kernel(
        body,
        out_type=[jax.ShapeDtypeStruct((NC, NPA, 128), jnp.float32)],
        mesh=_mesh,
        scratch_types=[
            pltpu.VMEM((CH,), jnp.int32),
            pltpu.VMEM((CH,), jnp.int32),
            pltpu.VMEM((CH, 16), jnp.float32),
            pltpu.VMEM((CH, 128), jnp.float32),
            pltpu.VMEM((CH, 128), jnp.float32),
            pltpu.VMEM((CH, 128), jnp.float32),
            pltpu.VMEM((CH, 128), jnp.float32),
            pltpu.VMEM((128,), jnp.float32),
            pltpu.VMEM_SHARED((NPA, 128), jnp.float32),
            pltpu.SemaphoreType.DMA,
            pltpu.SemaphoreType.DMA,
            pltpu.SemaphoreType.DMA,
            pltpu.SemaphoreType.DMA,
        ],
        compiler_params=_cparams,
    )
    return f(pc3, srcp, dstp, aE, bb)[0]


def _k4_body(x_ref, b_ref, o_ref):
    x = x_ref[...] + b_ref[...]
    m = jnp.max(x, axis=1, keepdims=True)
    ex = jnp.exp(x - m)
    o_ref[...] = ex / jnp.sum(ex, axis=1, keepdims=True)


def _softmax_out(x, b):
    return pl.pallas_call(
        _k4_body,
        grid=(G,),
        in_specs=[
            pl.BlockSpec((R, D), lambda i: (i, 0)),
            pl.BlockSpec((1, D), lambda i: (0, 0)),
        ],
        out_specs=pl.BlockSpec((R, D), lambda i: (i, 0)),
        out_shape=jax.ShapeDtypeStruct((NP, D), jnp.float32),
    )(x, b)


def _gat_layer(x, srcp, dstp, W, a_s, a_t, bias, H, F, do_elu, scale):
    proj, s2, cm = _proj_scores(x, W, a_s, a_t, H, F)
    c = cm[:, :H] + cm[:, H:]
    c = jnp.where(c > 0, c, NEG * c)
    cb = jnp.zeros((16,), jnp.float32).at[:H].set(c[0])
    sA = jnp.zeros((NP, 128), jnp.float32).at[:, :H].set(s2[:, :H])
    sB = jnp.zeros((NP, 128), jnp.float32).at[:, :H].set(s2[:, H:])
    pE, part = _edge_p(H, sA, sB, srcp, dstp, cb)
    nacc = NP if H == 1 else NPA
    df = _denom_reduce(H, part).reshape(nacc, H)
    dp = jnp.zeros((NP, 128), jnp.float32).at[:nacc, :H].set(df)
    aE = _edge_alpha(H, pE, dstp, dp)
    pc3 = proj.reshape(NP, H, NC, 128).transpose(1, 2, 0, 3).reshape(
        H * NC, NP, 128)
    bb = bias if do_elu else jnp.zeros((NC * 128,), jnp.float32)
    outT = _aggregate(H, do_elu, scale, pc3, srcp, dstp, aE, bb)
    o = outT.transpose(1, 0, 2).reshape(NPA, 256)
    return jnp.concatenate(
        [o, jnp.zeros((NP - NPA, 256), jnp.float32)], axis=0)


def kernel(graph, edge_index, W1, a_src1, a_tgt1, b1, W2, a_src2, a_tgt2, b2):
    xp = jnp.zeros((NP, D), jnp.float32).at[:N].set(graph)
    src0, dst0 = edge_index[0], edge_index[1]
    dstm = jnp.where(src0 != dst0, dst0, TRASH)
    loop = jnp.arange(N, dtype=jnp.int32)
    padi = jnp.zeros((EP - E2,), jnp.int32)
    srcp = jnp.concatenate([src0, loop, padi])
    dstp = jnp.concatenate([dstm, loop, padi + TRASH])

    h1 = _gat_layer(xp, srcp, dstp, W1, a_src1, a_tgt1, b1,
                    1, 256, do_elu=True, scale=1.0)
    agg2 = _gat_layer(h1, srcp, dstp, W2, a_src2, a_tgt2, b2,
                      8, 256, do_elu=False, scale=0.125)
    out = _softmax_out(agg2, b2.reshape(1, D))
    return out[:N]
